# baseline (speedup 1.0000x reference)
import functools

import jax
import jax.numpy as jnp
from jax import lax
from jax.experimental import pallas as pl
from jax.experimental.pallas import tpu as pltpu
from jax.experimental.pallas import tpu_sc as plsc

_NUM_BITS = 8
_THR = 4.0
_B, _F, _H, _K, _C = 1024, 128, 4096, 16, 10
_E = _F * _NUM_BITS

_HB = 512
_NW = 32


def _encode_body(x_ref, code_ref):
    x = x_ref[...]
    levels = jnp.round(jnp.clip(x, 0.0, 1.0) * (2 ** _NUM_BITS - 1)).astype(jnp.int32)
    gray = levels ^ (levels >> 1)
    parts = [((gray >> j) & 1).astype(jnp.bfloat16) for j in range(_NUM_BITS)]
    code_ref[...] = jnp.concatenate(parts, axis=1)


def _nt(a, m):
    return lax.dot_general(
        a, m, (((1,), (1,)), ((), ())), preferred_element_type=jnp.float32)


def _mm1_body(code_ref, m1_ref, oc0_ref, a1_ref, out1_ref):
    c = pl.program_id(0)

    @pl.when(c == 0)
    def _():
        out1_ref[...] = jnp.zeros_like(out1_ref)

    a1 = (_nt(code_ref[...], m1_ref[...].astype(jnp.bfloat16))
          >= _THR).astype(jnp.bfloat16)
    a1_ref[...] = a1
    out1_ref[...] += jnp.dot(a1.astype(jnp.float32), oc0_ref[...],
                             preferred_element_type=jnp.float32)


def _mm2_body(a1_ref, m2_ref, oc1_ref, out1_ref, out_ref):
    c = pl.program_id(0)

    @pl.when(c == 0)
    def _():
        out_ref[...] = out1_ref[...]

    z2 = _nt(a1_ref[...], m2_ref[...].astype(jnp.bfloat16))
    a2 = (z2 >= _THR).astype(jnp.float32)
    out_ref[...] += jnp.dot(a2, oc1_ref[...],
                            preferred_element_type=jnp.float32)


def _densify_sc_body(idx_hbm, w_hbm, out_hbm, buf, idx_v, w_v,
                     *, rows, chunk, permute, h_lo, h_hi):
    wid = lax.axis_index("s") * 2 + lax.axis_index("c")
    lane = lax.iota(jnp.int32, 16)

    for r in range(chunk):
        def zero_body(j, _, r=r):
            buf[r, pl.ds(j * 16, 16)] = jnp.zeros((16,), jnp.float32)
            return 0
        lax.fori_loop(0, rows // 16, zero_body, 0, unroll=8)

    rows_per_worker = (h_hi - h_lo) // _NW
    n_chunks = rows_per_worker // chunk
    groups = chunk // 16

    stage = max(rows_per_worker, 128)
    base = pl.multiple_of(h_lo + (wid * rows_per_worker // stage) * stage, 128)
    off = wid * rows_per_worker - (base - h_lo)
    pltpu.sync_copy(idx_hbm.at[:, pl.ds(base, stage)], idx_v)
    pltpu.sync_copy(w_hbm.at[:, pl.ds(base, stage)], w_v)

    def scatter_all(c, sign):
        for k in range(_K):
            for g in range(groups):
                ev = idx_v[k, pl.ds(off + c * chunk + g * 16, 16)]
                if permute:
                    ev = (ev & 7) * 128 + (ev >> 3)
                wv = w_v[k, pl.ds(off + c * chunk + g * 16, 16)]
                plsc.addupdate_scatter(buf, [lane + g * 16, ev], sign * wv)

    for c in range(n_chunks):
        h0 = wid * rows_per_worker + c * chunk
        scatter_all(c, 1.0)
        pltpu.sync_copy(buf, out_hbm.at[pl.ds(h0, chunk), :])
        if c != n_chunks - 1:
            scatter_all(c, -1.0)


def _make_densify(rows, chunk, permute, h_lo=0, h_hi=_H):
    mesh = plsc.VectorSubcoreMesh(core_axis_name="c", subcore_axis_name="s")
    return pl.kernel(
        functools.partial(_densify_sc_body, rows=rows, chunk=chunk,
                          permute=permute, h_lo=h_lo, h_hi=h_hi),
        mesh=mesh,
        compiler_params=pltpu.CompilerParams(needs_layout_passes=False),
        out_type=jax.ShapeDtypeStruct((h_hi - h_lo, rows), jnp.float32),
        scratch_types=[
            pltpu.VMEM((chunk, rows), jnp.float32),
            pltpu.VMEM((_K, max((h_hi - h_lo) // _NW, 128)), jnp.int32),
            pltpu.VMEM((_K, max((h_hi - h_lo) // _NW, 128)), jnp.float32),
        ],
    )


def kernel(x, idx1, w1, idx2, w2, out_conn):
    m1 = _make_densify(_E, 64, True)(idx1.T, w1.T)
    m2 = _make_densify(_H, 16, False)(idx2.T, w2.T)

    code = pl.pallas_call(
        _encode_body,
        out_shape=jax.ShapeDtypeStruct((_B, _E), jnp.bfloat16),
    )(x)

    nblk = _H // _HB
    a1, out1 = pl.pallas_call(
        _mm1_body,
        grid=(nblk,),
        in_specs=[
            pl.BlockSpec((_B, _E), lambda c: (0, 0)),
            pl.BlockSpec((_HB, _E), lambda c: (c, 0)),
            pl.BlockSpec((_HB, _C), lambda c: (c, 0)),
        ],
        out_specs=[
            pl.BlockSpec((_B, _HB), lambda c: (0, c)),
            pl.BlockSpec((_B, _C), lambda c: (0, 0)),
        ],
        out_shape=[
            jax.ShapeDtypeStruct((_B, _H), jnp.bfloat16),
            jax.ShapeDtypeStruct((_B, _C), jnp.float32),
        ],
    )(code, m1, out_conn[0])

    out = pl.pallas_call(
        _mm2_body,
        grid=(nblk,),
        in_specs=[
            pl.BlockSpec((_B, _H), lambda c: (0, 0)),
            pl.BlockSpec((_HB, _H), lambda c: (c, 0)),
            pl.BlockSpec((_HB, _C), lambda c: (c, 0)),
            pl.BlockSpec((_B, _C), lambda c: (0, 0)),
        ],
        out_specs=pl.BlockSpec((_B, _C), lambda c: (0, 0)),
        out_shape=jax.ShapeDtypeStruct((_B, _C), jnp.float32),
    )(a1, m2, out_conn[1], out1)

    return out


# speedup vs baseline: 1.0285x; 1.0285x over previous
import functools

import jax
import jax.numpy as jnp
from jax import lax
from jax.experimental import pallas as pl
from jax.experimental.pallas import tpu as pltpu
from jax.experimental.pallas import tpu_sc as plsc

_NUM_BITS = 8
_THR = 4.0
_B, _F, _H, _K, _C = 1024, 128, 4096, 16, 10
_E = _F * _NUM_BITS

_HB = 512
_NW = 32


def _encode_body(x_ref, code_ref):
    x = x_ref[...]
    levels = jnp.round(jnp.clip(x, 0.0, 1.0) * (2 ** _NUM_BITS - 1)).astype(jnp.int32)
    gray = levels ^ (levels >> 1)
    parts = [((gray >> j) & 1).astype(jnp.bfloat16) for j in range(_NUM_BITS)]
    code_ref[...] = jnp.concatenate(parts, axis=1)


def _nt(a, m):
    return lax.dot_general(
        a, m, (((1,), (1,)), ((), ())), preferred_element_type=jnp.float32)


def _mm1_body(code_ref, m1_ref, oc0_ref, a1_ref, out1_ref):
    c = pl.program_id(0)

    @pl.when(c == 0)
    def _():
        out1_ref[...] = jnp.zeros_like(out1_ref)

    a1 = (_nt(code_ref[...], m1_ref[...].astype(jnp.bfloat16))
          >= _THR).astype(jnp.bfloat16)
    a1_ref[...] = a1
    out1_ref[...] += jnp.dot(a1.astype(jnp.float32), oc0_ref[...],
                             preferred_element_type=jnp.float32)


def _mm2_body(a1_ref, m2_ref, oc1_ref, out1_ref, out_ref):
    c = pl.program_id(0)

    @pl.when(c == 0)
    def _():
        out_ref[...] = out1_ref[...]

    z2 = _nt(a1_ref[...], m2_ref[...].astype(jnp.bfloat16))
    a2 = (z2 >= _THR).astype(jnp.float32)
    out_ref[...] += jnp.dot(a2, oc1_ref[...],
                            preferred_element_type=jnp.float32)


def _densify_sc_body(idx_hbm, w_hbm, out_hbm, buf, idx_v, w_v,
                     *, rows, chunk, permute, h_lo, h_hi):
    wid = lax.axis_index("s") * 2 + lax.axis_index("c")
    lane = lax.iota(jnp.int32, 16)

    for r in range(chunk):
        def zero_body(j, _, r=r):
            buf[r, pl.ds(j * 16, 16)] = jnp.zeros((16,), jnp.float32)
            return 0
        lax.fori_loop(0, rows // 16, zero_body, 0, unroll=8)

    rows_per_worker = (h_hi - h_lo) // _NW
    n_chunks = rows_per_worker // chunk
    groups = chunk // 16

    stage = max(rows_per_worker, 128)
    base = pl.multiple_of(h_lo + (wid * rows_per_worker // stage) * stage, 128)
    off = wid * rows_per_worker - (base - h_lo)
    pltpu.sync_copy(idx_hbm.at[:, pl.ds(base, stage)], idx_v)
    pltpu.sync_copy(w_hbm.at[:, pl.ds(base, stage)], w_v)

    def scatter_all(c, sign):
        for k in range(_K):
            for g in range(groups):
                ev = idx_v[k, pl.ds(off + c * chunk + g * 16, 16)]
                if permute:
                    ev = (ev & 7) * 128 + (ev >> 3)
                wv = w_v[k, pl.ds(off + c * chunk + g * 16, 16)]
                plsc.addupdate_scatter(buf, [lane + g * 16, ev], sign * wv)

    for c in range(n_chunks):
        h0 = wid * rows_per_worker + c * chunk
        scatter_all(c, 1.0)
        pltpu.sync_copy(buf, out_hbm.at[pl.ds(h0, chunk), :])
        if c != n_chunks - 1:
            scatter_all(c, -1.0)


def _make_densify(rows, chunk, permute, h_lo=0, h_hi=_H):
    mesh = plsc.VectorSubcoreMesh(core_axis_name="c", subcore_axis_name="s")
    return pl.kernel(
        functools.partial(_densify_sc_body, rows=rows, chunk=chunk,
                          permute=permute, h_lo=h_lo, h_hi=h_hi),
        mesh=mesh,
        compiler_params=pltpu.CompilerParams(needs_layout_passes=False),
        out_type=jax.ShapeDtypeStruct((h_hi - h_lo, rows), jnp.float32),
        scratch_types=[
            pltpu.VMEM((chunk, rows), jnp.float32),
            pltpu.VMEM((_K, max((h_hi - h_lo) // _NW, 128)), jnp.int32),
            pltpu.VMEM((_K, max((h_hi - h_lo) // _NW, 128)), jnp.float32),
        ],
    )


def kernel(x, idx1, w1, idx2, w2, out_conn):
    m1 = _make_densify(_E, 32, True)(idx1.T, w1.T)
    m2 = _make_densify(_H, 16, False)(idx2.T, w2.T)

    code = pl.pallas_call(
        _encode_body,
        out_shape=jax.ShapeDtypeStruct((_B, _E), jnp.bfloat16),
    )(x)

    nblk = _H // _HB
    a1, out1 = pl.pallas_call(
        _mm1_body,
        grid=(nblk,),
        in_specs=[
            pl.BlockSpec((_B, _E), lambda c: (0, 0)),
            pl.BlockSpec((_HB, _E), lambda c: (c, 0)),
            pl.BlockSpec((_HB, _C), lambda c: (c, 0)),
        ],
        out_specs=[
            pl.BlockSpec((_B, _HB), lambda c: (0, c)),
            pl.BlockSpec((_B, _C), lambda c: (0, 0)),
        ],
        out_shape=[
            jax.ShapeDtypeStruct((_B, _H), jnp.bfloat16),
            jax.ShapeDtypeStruct((_B, _C), jnp.float32),
        ],
    )(code, m1, out_conn[0])

    out = pl.pallas_call(
        _mm2_body,
        grid=(nblk,),
        in_specs=[
            pl.BlockSpec((_B, _H), lambda c: (0, 0)),
            pl.BlockSpec((_HB, _H), lambda c: (c, 0)),
            pl.BlockSpec((_HB, _C), lambda c: (c, 0)),
            pl.BlockSpec((_B, _C), lambda c: (0, 0)),
        ],
        out_specs=pl.BlockSpec((_B, _C), lambda c: (0, 0)),
        out_shape=jax.ShapeDtypeStruct((_B, _C), jnp.float32),
    )(a1, m2, out_conn[1], out1)

    return out


# speedup vs baseline: 1.0286x; 1.0001x over previous
import functools

import jax
import jax.numpy as jnp
from jax import lax
from jax.experimental import pallas as pl
from jax.experimental.pallas import tpu as pltpu
from jax.experimental.pallas import tpu_sc as plsc

_NUM_BITS = 8
_THR = 4.0
_B, _F, _H, _K, _C = 1024, 128, 4096, 16, 10
_E = _F * _NUM_BITS

_HB = 512
_NW = 32


def _encode_body(x_ref, code_ref):
    x = x_ref[...]
    levels = jnp.round(jnp.clip(x, 0.0, 1.0) * (2 ** _NUM_BITS - 1)).astype(jnp.int32)
    gray = levels ^ (levels >> 1)
    parts = [((gray >> j) & 1).astype(jnp.bfloat16) for j in range(_NUM_BITS)]
    code_ref[...] = jnp.concatenate(parts, axis=1)


def _nt(a, m):
    return lax.dot_general(
        a, m, (((1,), (1,)), ((), ())), preferred_element_type=jnp.float32)


def _mm1_body(code_ref, m1_ref, oc0_ref, a1_ref, out1_ref):
    c = pl.program_id(0)

    @pl.when(c == 0)
    def _():
        out1_ref[...] = jnp.zeros_like(out1_ref)

    a1 = (_nt(code_ref[...], m1_ref[...].astype(jnp.bfloat16))
          >= _THR).astype(jnp.bfloat16)
    a1_ref[...] = a1
    out1_ref[...] += jnp.dot(a1.astype(jnp.float32), oc0_ref[...],
                             preferred_element_type=jnp.float32)


def _mm2_body(a1_ref, m2_ref, oc1_ref, out1_ref, out_ref):
    c = pl.program_id(0)

    @pl.when(c == 0)
    def _():
        out_ref[...] = out1_ref[...]

    z2 = _nt(a1_ref[...].astype(jnp.float32), m2_ref[...])
    a2 = (z2 >= _THR).astype(jnp.float32)
    out_ref[...] += jnp.dot(a2, oc1_ref[...],
                            preferred_element_type=jnp.float32)


def _densify_sc_body(idx_hbm, w_hbm, out_hbm, buf, idx_v, w_v,
                     *, rows, chunk, permute, h_lo, h_hi):
    wid = lax.axis_index("s") * 2 + lax.axis_index("c")
    lane = lax.iota(jnp.int32, 16)

    for r in range(chunk):
        def zero_body(j, _, r=r):
            buf[r, pl.ds(j * 16, 16)] = jnp.zeros((16,), jnp.float32)
            return 0
        lax.fori_loop(0, rows // 16, zero_body, 0, unroll=8)

    rows_per_worker = (h_hi - h_lo) // _NW
    n_chunks = rows_per_worker // chunk
    groups = chunk // 16

    stage = max(rows_per_worker, 128)
    base = pl.multiple_of(h_lo + (wid * rows_per_worker // stage) * stage, 128)
    off = wid * rows_per_worker - (base - h_lo)
    pltpu.sync_copy(idx_hbm.at[:, pl.ds(base, stage)], idx_v)
    pltpu.sync_copy(w_hbm.at[:, pl.ds(base, stage)], w_v)

    def scatter_all(c, sign):
        for k in range(_K):
            for g in range(groups):
                ev = idx_v[k, pl.ds(off + c * chunk + g * 16, 16)]
                if permute:
                    ev = (ev & 7) * 128 + (ev >> 3)
                wv = w_v[k, pl.ds(off + c * chunk + g * 16, 16)]
                plsc.addupdate_scatter(buf, [lane + g * 16, ev], sign * wv)

    for c in range(n_chunks):
        h0 = wid * rows_per_worker + c * chunk
        scatter_all(c, 1.0)
        pltpu.sync_copy(buf, out_hbm.at[pl.ds(h0, chunk), :])
        if c != n_chunks - 1:
            scatter_all(c, -1.0)


def _make_densify(rows, chunk, permute, h_lo=0, h_hi=_H):
    mesh = plsc.VectorSubcoreMesh(core_axis_name="c", subcore_axis_name="s")
    return pl.kernel(
        functools.partial(_densify_sc_body, rows=rows, chunk=chunk,
                          permute=permute, h_lo=h_lo, h_hi=h_hi),
        mesh=mesh,
        compiler_params=pltpu.CompilerParams(needs_layout_passes=False),
        out_type=jax.ShapeDtypeStruct((h_hi - h_lo, rows), jnp.float32),
        scratch_types=[
            pltpu.VMEM((chunk, rows), jnp.float32),
            pltpu.VMEM((_K, max((h_hi - h_lo) // _NW, 128)), jnp.int32),
            pltpu.VMEM((_K, max((h_hi - h_lo) // _NW, 128)), jnp.float32),
        ],
    )


def kernel(x, idx1, w1, idx2, w2, out_conn):
    m1 = _make_densify(_E, 32, True)(idx1.T, w1.T)
    m2 = _make_densify(_H, 16, False)(idx2.T, w2.T)

    code = pl.pallas_call(
        _encode_body,
        out_shape=jax.ShapeDtypeStruct((_B, _E), jnp.bfloat16),
    )(x)

    nblk = _H // _HB
    a1, out1 = pl.pallas_call(
        _mm1_body,
        grid=(nblk,),
        in_specs=[
            pl.BlockSpec((_B, _E), lambda c: (0, 0)),
            pl.BlockSpec((_HB, _E), lambda c: (c, 0)),
            pl.BlockSpec((_HB, _C), lambda c: (c, 0)),
        ],
        out_specs=[
            pl.BlockSpec((_B, _HB), lambda c: (0, c)),
            pl.BlockSpec((_B, _C), lambda c: (0, 0)),
        ],
        out_shape=[
            jax.ShapeDtypeStruct((_B, _H), jnp.bfloat16),
            jax.ShapeDtypeStruct((_B, _C), jnp.float32),
        ],
    )(code, m1, out_conn[0])

    out = pl.pallas_call(
        _mm2_body,
        grid=(nblk,),
        in_specs=[
            pl.BlockSpec((_B, _H), lambda c: (0, 0)),
            pl.BlockSpec((_HB, _H), lambda c: (c, 0)),
            pl.BlockSpec((_HB, _C), lambda c: (c, 0)),
            pl.BlockSpec((_B, _C), lambda c: (0, 0)),
        ],
        out_specs=pl.BlockSpec((_B, _C), lambda c: (0, 0)),
        out_shape=jax.ShapeDtypeStruct((_B, _C), jnp.float32),
    )(a1, m2, out_conn[1], out1)

    return out


# speedup vs baseline: 1.0292x; 1.0006x over previous
import functools

import jax
import jax.numpy as jnp
from jax import lax
from jax.experimental import pallas as pl
from jax.experimental.pallas import tpu as pltpu
from jax.experimental.pallas import tpu_sc as plsc

_NUM_BITS = 8
_THR = 4.0
_B, _F, _H, _K, _C = 1024, 128, 4096, 16, 10
_E = _F * _NUM_BITS

_HB = 512
_NW = 32


def _encode_body(x_ref, code_ref):
    x = x_ref[...]
    levels = jnp.round(jnp.clip(x, 0.0, 1.0) * (2 ** _NUM_BITS - 1)).astype(jnp.int32)
    gray = levels ^ (levels >> 1)
    parts = [((gray >> j) & 1).astype(jnp.bfloat16) for j in range(_NUM_BITS)]
    code_ref[...] = jnp.concatenate(parts, axis=1)


def _nt(a, m):
    return lax.dot_general(
        a, m, (((1,), (1,)), ((), ())), preferred_element_type=jnp.float32)


def _mm1_body(code_ref, m1_ref, oc0_ref, a1_ref, out1_ref):
    c = pl.program_id(0)

    @pl.when(c == 0)
    def _():
        out1_ref[...] = jnp.zeros_like(out1_ref)

    a1 = (_nt(code_ref[...], m1_ref[...].astype(jnp.bfloat16))
          >= _THR).astype(jnp.bfloat16)
    a1_ref[...] = a1
    out1_ref[...] += jnp.dot(a1.astype(jnp.float32), oc0_ref[...],
                             preferred_element_type=jnp.float32)


def _mm2_body(a1_ref, m2_ref, oc1_ref, out1_ref, out_ref):
    c = pl.program_id(0)

    @pl.when(c == 0)
    def _():
        out_ref[...] = out1_ref[...]

    z2 = _nt(a1_ref[...], m2_ref[...].astype(jnp.bfloat16))
    a2 = (z2 >= _THR).astype(jnp.float32)
    out_ref[...] += jnp.dot(a2, oc1_ref[...],
                            preferred_element_type=jnp.float32)


def _densify_sc_body(idx_hbm, w_hbm, out_hbm, buf, idx_v, w_v,
                     *, rows, chunk, permute, h_lo, h_hi):
    wid = lax.axis_index("s") * 2 + lax.axis_index("c")
    lane = lax.iota(jnp.int32, 16)

    for r in range(chunk):
        def zero_body(j, _, r=r):
            buf[r, pl.ds(j * 16, 16)] = jnp.zeros((16,), jnp.float32)
            return 0
        lax.fori_loop(0, rows // 16, zero_body, 0, unroll=8)

    rows_per_worker = (h_hi - h_lo) // _NW
    n_chunks = rows_per_worker // chunk
    groups = chunk // 16

    stage = max(rows_per_worker, 128)
    base = pl.multiple_of(h_lo + (wid * rows_per_worker // stage) * stage, 128)
    off = wid * rows_per_worker - (base - h_lo)
    pltpu.sync_copy(idx_hbm.at[:, pl.ds(base, stage)], idx_v)
    pltpu.sync_copy(w_hbm.at[:, pl.ds(base, stage)], w_v)

    def scatter_all(c, sign):
        for k in range(_K):
            for g in range(groups):
                ev = idx_v[k, pl.ds(off + c * chunk + g * 16, 16)]
                if permute:
                    ev = (ev & 7) * 128 + (ev >> 3)
                wv = w_v[k, pl.ds(off + c * chunk + g * 16, 16)]
                plsc.addupdate_scatter(buf, [lane + g * 16, ev], sign * wv)

    for c in range(n_chunks):
        h0 = wid * rows_per_worker + c * chunk
        scatter_all(c, 1.0)
        pltpu.sync_copy(buf, out_hbm.at[pl.ds(h0, chunk), :])
        if c != n_chunks - 1:
            scatter_all(c, -1.0)


def _make_densify(rows, chunk, permute, h_lo=0, h_hi=_H):
    mesh = plsc.VectorSubcoreMesh(core_axis_name="c", subcore_axis_name="s")
    return pl.kernel(
        functools.partial(_densify_sc_body, rows=rows, chunk=chunk,
                          permute=permute, h_lo=h_lo, h_hi=h_hi),
        mesh=mesh,
        compiler_params=pltpu.CompilerParams(needs_layout_passes=False),
        out_type=jax.ShapeDtypeStruct((h_hi - h_lo, rows), jnp.float32),
        scratch_types=[
            pltpu.VMEM((chunk, rows), jnp.float32),
            pltpu.VMEM((_K, max((h_hi - h_lo) // _NW, 128)), jnp.int32),
            pltpu.VMEM((_K, max((h_hi - h_lo) // _NW, 128)), jnp.float32),
        ],
    )


def kernel(x, idx1, w1, idx2, w2, out_conn):
    m1 = _make_densify(_E, 32, True)(idx1.T, w1.T)
    m2 = _make_densify(_H, 16, False)(idx2.T, w2.T)

    code = pl.pallas_call(
        _encode_body,
        out_shape=jax.ShapeDtypeStruct((_B, _E), jnp.bfloat16),
    )(x)

    nblk = _H // _HB
    a1, out1 = pl.pallas_call(
        _mm1_body,
        grid=(nblk,),
        in_specs=[
            pl.BlockSpec((_B, _E), lambda c: (0, 0)),
            pl.BlockSpec((_HB, _E), lambda c: (c, 0)),
            pl.BlockSpec((_HB, _C), lambda c: (c, 0)),
        ],
        out_specs=[
            pl.BlockSpec((_B, _HB), lambda c: (0, c)),
            pl.BlockSpec((_B, _C), lambda c: (0, 0)),
        ],
        out_shape=[
            jax.ShapeDtypeStruct((_B, _H), jnp.bfloat16),
            jax.ShapeDtypeStruct((_B, _C), jnp.float32),
        ],
    )(code, m1, out_conn[0])

    out = pl.pallas_call(
        _mm2_body,
        grid=(nblk,),
        in_specs=[
            pl.BlockSpec((_B, _H), lambda c: (0, 0)),
            pl.BlockSpec((_HB, _H), lambda c: (c, 0)),
            pl.BlockSpec((_HB, _C), lambda c: (c, 0)),
            pl.BlockSpec((_B, _C), lambda c: (0, 0)),
        ],
        out_specs=pl.BlockSpec((_B, _C), lambda c: (0, 0)),
        out_shape=jax.ShapeDtypeStruct((_B, _C), jnp.float32),
    )(a1, m2, out_conn[1], out1)

    return out
